# baseline (speedup 1.0000x reference)
import functools

import jax
import jax.numpy as jnp
from jax import lax
from jax.experimental import pallas as pl
from jax.experimental.pallas import tpu as pltpu
from jax.experimental.pallas import tpu_sc as plsc

N = 10000
D = 256
H = 256
T = 128
G = 64

NC = 2
NS = 16
LANES = 16

CHUNK = 128
E_PAD = NC * NS * CHUNK * 40
GARBAGE = N
NR = 10240
ROWS_PER_TILE = NR // NS

BR = 1000

_sc_mesh = plsc.VectorSubcoreMesh(core_axis_name="c", subcore_axis_name="s")


NCH_CNT = E_PAD // (NC * NS) // CHUNK
NBUF = 2
CH4 = 128
NCH4 = E_PAD // NS // CH4
NPHASE = 2


@functools.partial(
    pl.kernel,
    out_type=jax.ShapeDtypeStruct((NC * NR, LANES), jnp.float32),
    mesh=_sc_mesh,
    scratch_types=[
        pltpu.VMEM((NCH_CNT, CHUNK), jnp.int32),
        pltpu.VMEM((CHUNK, LANES), jnp.float32),
        pltpu.VMEM((CHUNK, LANES), jnp.float32),
        pltpu.VMEM_SHARED((NR, LANES), jnp.float32),
        pltpu.SemaphoreType.DMA,
    ],
)
def _count_kernel(dst2_hbm, out_hbm, idx_v, ones_v, zero_v, acc, sem):
    c = lax.axis_index("c")
    s = lax.axis_index("s")
    zvec = jnp.zeros((LANES,), jnp.float32)
    onevec = jnp.where(lax.iota(jnp.int32, LANES) == 0, 1.0, 0.0)

    def fill(r, carry):
        ones_v[r, :] = onevec
        zero_v[r, :] = zvec
        return carry

    lax.fori_loop(0, CHUNK, fill, 0)

    crow = (c * NS + s) * NCH_CNT
    pltpu.sync_copy(dst2_hbm.at[pl.ds(crow, NCH_CNT)], idx_v)

    row0 = s * ROWS_PER_TILE
    for k in range(ROWS_PER_TILE // CHUNK):
        pltpu.sync_copy(zero_v, acc.at[pl.ds(row0 + k * CHUNK, CHUNK)])
    plsc.subcore_barrier()

    def body(k, carry):
        pltpu.async_copy(ones_v, acc.at[idx_v.at[k]], sem, add=True)
        return carry

    lax.fori_loop(0, NCH_CNT, body, 0)

    def drain(k, carry):
        pltpu.make_async_copy(ones_v, acc.at[idx_v.at[0]], sem).wait()
        return carry

    lax.fori_loop(0, NCH_CNT, drain, 0)
    plsc.subcore_barrier()

    for k in range(ROWS_PER_TILE // CHUNK):
        pltpu.sync_copy(acc.at[pl.ds(row0 + k * CHUNK, CHUNK)],
                        out_hbm.at[pl.ds(c * NR + row0 + k * CHUNK, CHUNK)])


@functools.partial(
    pl.kernel,
    out_type=jax.ShapeDtypeStruct((NC * NR, T), jnp.float32),
    mesh=_sc_mesh,
    scratch_types=[
        pltpu.VMEM((NCH4 // NPHASE, CH4), jnp.int32),
        pltpu.VMEM((NCH4 // NPHASE, CH4), jnp.int32),
        [pltpu.VMEM((CH4, T), jnp.float32) for _ in range(NBUF)],
        pltpu.VMEM_SHARED((NR, T), jnp.float32),
        [pltpu.SemaphoreType.DMA for _ in range(NBUF)],
        [pltpu.SemaphoreType.DMA for _ in range(NBUF)],
    ],
)
def _spmm_kernel(table_hbm, src4_hbm, dst4_hbm, out_hbm,
                 isrc, idst, rows, acc, g, s_sem):
    c = lax.axis_index("c")
    s = lax.axis_index("s")
    zvec = jnp.zeros((LANES,), jnp.float32)
    HALF = NCH4 // NPHASE

    def zfill(r, carry):
        for j in range(T // LANES):
            rows[0][r, pl.ds(j * LANES, LANES)] = zvec
        return carry

    lax.fori_loop(0, CH4, zfill, 0)

    row0 = s * ROWS_PER_TILE
    for k in range(ROWS_PER_TILE // CH4):
        pltpu.sync_copy(rows[0], acc.at[pl.ds(row0 + k * CH4, CH4)])
    plsc.subcore_barrier()

    coff = c * N

    for phase in range(NPHASE):
        crow = s * NCH4 + phase * HALF
        pltpu.sync_copy(src4_hbm.at[pl.ds(crow, HALF)], isrc)
        pltpu.sync_copy(dst4_hbm.at[pl.ds(crow, HALF)], idst)

        def offadd(r, carry):
            for j in range(CH4 // LANES):
                sl = pl.ds(j * LANES, LANES)
                isrc[r, sl] = isrc[r, sl] + coff
            return carry

        lax.fori_loop(0, HALF, offadd, 0)

        for b in range(NBUF):
            pltpu.async_copy(table_hbm.at[isrc.at[b]], rows[b], g[b])

        def body(m, carry):
            for b in range(NBUF):
                k = NBUF * m + b
                pltpu.make_async_copy(
                    table_hbm.at[isrc.at[k]], rows[b], g[b]).wait()
                pltpu.async_copy(rows[b], acc.at[idst.at[k]], s_sem[b],
                                 add=True)

                @pl.when(k + NBUF < HALF)
                def _():
                    pltpu.make_async_copy(
                        rows[b], acc.at[idst.at[k]], s_sem[b]).wait()
                    pltpu.async_copy(
                        table_hbm.at[isrc.at[k + NBUF]], rows[b], g[b])

            return carry

        lax.fori_loop(0, HALF // NBUF, body, 0)
        for b in range(NBUF):
            pltpu.make_async_copy(rows[b], acc.at[idst.at[0]], s_sem[b]).wait()
    plsc.subcore_barrier()

    for k in range(ROWS_PER_TILE // CHUNK):
        pltpu.sync_copy(acc.at[pl.ds(row0 + k * CHUNK, CHUNK)],
                        out_hbm.at[pl.ds(c * NR + row0 + k * CHUNK, CHUNK)])


def _dis_from_counts(counts_ref):
    cnt = counts_ref[0, :, 0:1] + counts_ref[1, :, 0:1]
    return lax.rsqrt(cnt + 1.0)


def _mm_scale_body(counts_ref, x_ref, w_ref, out_ref):
    dis = _dis_from_counts(counts_ref)
    p = jnp.dot(x_ref[...], w_ref[...], preferred_element_type=jnp.float32) * dis
    out_ref[0] = p[:, :T]
    out_ref[1] = p[:, T:]


def _finish_stats_body(counts_ref, s_ref, p_ref, b_ref, a_ref, stats_ref):
    dis = _dis_from_counts(counts_ref)
    a = jnp.concatenate([s_ref[0] + p_ref[0], s_ref[1] + p_ref[1]], axis=1)
    a = a * dis + b_ref[...]
    a_ref[...] = a

    @pl.when(pl.program_id(0) == 0)
    def _():
        stats_ref[...] = jnp.zeros_like(stats_ref)

    stats_ref[0:1, :] = stats_ref[0:1, :] + jnp.sum(a, axis=0, keepdims=True)
    stats_ref[1:2, :] = stats_ref[1:2, :] + jnp.sum(a * a, axis=0, keepdims=True)


def _bn_mm_body(counts_ref, a_ref, stats_ref, g_ref, be_ref, w_ref, out_ref):
    dis = _dis_from_counts(counts_ref)
    m = stats_ref[0:1, :] / N
    v = stats_ref[1:2, :] / N - m * m
    h = (a_ref[...] - m) * lax.rsqrt(v + 1e-5) * g_ref[...] + be_ref[...]
    h = jnp.maximum(h, 0.0)
    p = jnp.dot(h, w_ref[...], preferred_element_type=jnp.float32) * dis
    out_ref[0] = p[:, :T]
    out_ref[1] = p[:, T:]


def _pool_steps(counts_ref, s_ref, p_ref, b_ref, batch_ref, psum_ref, pcnt_ref):
    dis = _dis_from_counts(counts_ref)
    a = jnp.concatenate([s_ref[0] + p_ref[0], s_ref[1] + p_ref[1]], axis=1)
    h = jnp.maximum(a * dis + b_ref[...], 0.0)
    oh = (batch_ref[...] == lax.broadcasted_iota(jnp.int32, (1, G), 1))
    oh = oh.astype(jnp.float32)

    @pl.when(pl.program_id(0) == 0)
    def _():
        psum_ref[...] = jnp.zeros_like(psum_ref)
        pcnt_ref[...] = jnp.zeros_like(pcnt_ref)

    dn = (((0,), (0,)), ((), ()))
    psum_ref[...] = psum_ref[...] + lax.dot_general(
        oh, h, dn, preferred_element_type=jnp.float32)
    pcnt_ref[...] = pcnt_ref[...] + lax.dot_general(
        oh, jnp.ones((BR, 8), jnp.float32), dn, preferred_element_type=jnp.float32)


def _final_step(psum_ref, pcnt_ref, wl_ref, bl_ref, out_ref):
    pooled = psum_ref[...] / jnp.maximum(pcnt_ref[:, 0:1], 1.0)
    out_ref[...] = jnp.dot(pooled, wl_ref[...],
                           preferred_element_type=jnp.float32) + bl_ref[...]


NB = N // BR

_counts_spec = pl.BlockSpec((2, BR, LANES), lambda i: (0, i, 0))
_half_spec = pl.BlockSpec((2, BR, T), lambda i: (0, i, 0))

_finish_stats = pl.pallas_call(
    _finish_stats_body,
    grid=(NB,),
    in_specs=[_counts_spec, _half_spec, _half_spec,
              pl.BlockSpec((1, H), lambda i: (0, 0))],
    out_specs=[pl.BlockSpec((BR, H), lambda i: (i, 0)),
               pl.BlockSpec((2, H), lambda i: (0, 0))],
    out_shape=[jax.ShapeDtypeStruct((N, H), jnp.float32),
               jax.ShapeDtypeStruct((2, H), jnp.float32)],
)

_bn_mm = pl.pallas_call(
    _bn_mm_body,
    grid=(NB,),
    in_specs=[_counts_spec, pl.BlockSpec((BR, H), lambda i: (i, 0)),
              pl.BlockSpec((2, H), lambda i: (0, 0)),
              pl.BlockSpec((1, H), lambda i: (0, 0)),
              pl.BlockSpec((1, H), lambda i: (0, 0)),
              pl.BlockSpec((H, H), lambda i: (0, 0))],
    out_specs=_half_spec,
    out_shape=jax.ShapeDtypeStruct((2, N, T), jnp.float32),
)

_mm_scale = pl.pallas_call(
    _mm_scale_body,
    grid=(NB,),
    in_specs=[_counts_spec, pl.BlockSpec((BR, D), lambda i: (i, 0)),
              pl.BlockSpec((D, H), lambda i: (0, 0))],
    out_specs=_half_spec,
    out_shape=jax.ShapeDtypeStruct((2, N, T), jnp.float32),
)

_pool = pl.pallas_call(
    lambda counts_ref, s_ref, p_ref, b_ref, batch_ref, psum_ref, pcnt_ref:
        _pool_steps(counts_ref, s_ref, p_ref, b_ref, batch_ref, psum_ref, pcnt_ref),
    grid=(NB,),
    in_specs=[_counts_spec, _half_spec, _half_spec,
              pl.BlockSpec((1, H), lambda i: (0, 0)),
              pl.BlockSpec((BR, 1), lambda i: (i, 0))],
    out_specs=[pl.BlockSpec((G, H), lambda i: (0, 0)),
               pl.BlockSpec((G, 8), lambda i: (0, 0))],
    out_shape=[jax.ShapeDtypeStruct((G, H), jnp.float32),
               jax.ShapeDtypeStruct((G, 8), jnp.float32)],
)

_final = pl.pallas_call(
    lambda psum_ref, pcnt_ref, wl_ref, bl_ref, out_ref:
        _final_step(psum_ref, pcnt_ref, wl_ref, bl_ref, out_ref),
    grid=(1,),
    in_specs=[pl.BlockSpec((G, H), lambda i: (0, 0)),
              pl.BlockSpec((G, 8), lambda i: (0, 0)),
              pl.BlockSpec((H, T), lambda i: (0, 0)),
              pl.BlockSpec((1, T), lambda i: (0, 0))],
    out_specs=pl.BlockSpec((G, T), lambda i: (0, 0)),
    out_shape=jax.ShapeDtypeStruct((G, T), jnp.float32),
)

def kernel(x, edge_index, batch, W1, b1, W2, b2, W3, b3, g1, be1, g2, be2, Wl, bl):
    src = edge_index[0]
    dst = edge_index[1]
    pad = E_PAD - src.shape[0]
    src_p = jnp.concatenate([src, jnp.zeros((pad,), src.dtype)])
    dst_p = jnp.concatenate([dst, jnp.full((pad,), GARBAGE, dst.dtype)])
    dst2 = dst_p.reshape(E_PAD // CHUNK, CHUNK)
    src4 = src_p.reshape(E_PAD // CH4, CH4)
    dst4 = dst_p.reshape(E_PAD // CH4, CH4)

    counts = _count_kernel(dst2).reshape(NC, NR, LANES)

    def spmm(p):
        return _spmm_kernel(p.reshape(NC * N, T), src4, dst4).reshape(NC, NR, T)

    b1r, b2r, b3r = b1.reshape(1, H), b2.reshape(1, H), b3.reshape(1, H)

    p1 = _mm_scale(counts, x, W1)
    s1 = spmm(p1)
    a1, st1 = _finish_stats(counts, s1, p1, b1r)
    p2 = _bn_mm(counts, a1, st1, g1.reshape(1, H), be1.reshape(1, H), W2)
    s2 = spmm(p2)
    a2, st2 = _finish_stats(counts, s2, p2, b2r)
    p3 = _bn_mm(counts, a2, st2, g2.reshape(1, H), be2.reshape(1, H), W3)
    s3 = spmm(p3)
    psum, pcnt = _pool(counts, s3, p3, b3r, batch.reshape(N, 1))
    return _final(psum, pcnt, Wl, bl.reshape(1, T))


# speedup vs baseline: 1.0088x; 1.0088x over previous
import functools

import jax
import jax.numpy as jnp
from jax import lax
from jax.experimental import pallas as pl
from jax.experimental.pallas import tpu as pltpu
from jax.experimental.pallas import tpu_sc as plsc

N = 10000
D = 256
H = 256
T = 128
G = 64

NC = 2
NS = 16
LANES = 16

CHUNK = 128
E_PAD = NC * NS * CHUNK * 40
GARBAGE = N
NR = 10240
ROWS_PER_TILE = NR // NS

BR = 2000

_sc_mesh = plsc.VectorSubcoreMesh(core_axis_name="c", subcore_axis_name="s")


NCH_CNT = E_PAD // (NC * NS) // CHUNK
NBUF = 2
CH4 = 128
NCH4 = E_PAD // NS // CH4
NPHASE = 2


@functools.partial(
    pl.kernel,
    out_type=jax.ShapeDtypeStruct((NC * NR, LANES), jnp.float32),
    mesh=_sc_mesh,
    scratch_types=[
        pltpu.VMEM((NCH_CNT, CHUNK), jnp.int32),
        pltpu.VMEM((CHUNK, LANES), jnp.float32),
        pltpu.VMEM((CHUNK, LANES), jnp.float32),
        pltpu.VMEM_SHARED((NR, LANES), jnp.float32),
        pltpu.SemaphoreType.DMA,
    ],
)
def _count_kernel(dst2_hbm, out_hbm, idx_v, ones_v, zero_v, acc, sem):
    c = lax.axis_index("c")
    s = lax.axis_index("s")
    zvec = jnp.zeros((LANES,), jnp.float32)
    onevec = jnp.where(lax.iota(jnp.int32, LANES) == 0, 1.0, 0.0)

    def fill(r, carry):
        ones_v[r, :] = onevec
        zero_v[r, :] = zvec
        return carry

    lax.fori_loop(0, CHUNK, fill, 0)

    crow = (c * NS + s) * NCH_CNT
    pltpu.sync_copy(dst2_hbm.at[pl.ds(crow, NCH_CNT)], idx_v)

    row0 = s * ROWS_PER_TILE
    for k in range(ROWS_PER_TILE // CHUNK):
        pltpu.sync_copy(zero_v, acc.at[pl.ds(row0 + k * CHUNK, CHUNK)])
    plsc.subcore_barrier()

    def body(k, carry):
        pltpu.async_copy(ones_v, acc.at[idx_v.at[k]], sem, add=True)
        return carry

    lax.fori_loop(0, NCH_CNT, body, 0)

    def drain(k, carry):
        pltpu.make_async_copy(ones_v, acc.at[idx_v.at[0]], sem).wait()
        return carry

    lax.fori_loop(0, NCH_CNT, drain, 0)
    plsc.subcore_barrier()

    for k in range(ROWS_PER_TILE // CHUNK):
        pltpu.sync_copy(acc.at[pl.ds(row0 + k * CHUNK, CHUNK)],
                        out_hbm.at[pl.ds(c * NR + row0 + k * CHUNK, CHUNK)])


@functools.partial(
    pl.kernel,
    out_type=jax.ShapeDtypeStruct((NC * NR, T), jnp.float32),
    mesh=_sc_mesh,
    scratch_types=[
        pltpu.VMEM((NCH4 // NPHASE, CH4), jnp.int32),
        pltpu.VMEM((NCH4 // NPHASE, CH4), jnp.int32),
        [pltpu.VMEM((CH4, T), jnp.float32) for _ in range(NBUF)],
        pltpu.VMEM_SHARED((NR, T), jnp.float32),
        [pltpu.SemaphoreType.DMA for _ in range(NBUF)],
        [pltpu.SemaphoreType.DMA for _ in range(NBUF)],
    ],
)
def _spmm_kernel(table_hbm, src4_hbm, dst4_hbm, out_hbm,
                 isrc, idst, rows, acc, g, s_sem):
    c = lax.axis_index("c")
    s = lax.axis_index("s")
    zvec = jnp.zeros((LANES,), jnp.float32)
    HALF = NCH4 // NPHASE

    def zfill(r, carry):
        for j in range(T // LANES):
            rows[0][r, pl.ds(j * LANES, LANES)] = zvec
        return carry

    lax.fori_loop(0, CH4, zfill, 0)

    row0 = s * ROWS_PER_TILE
    for k in range(ROWS_PER_TILE // CH4):
        pltpu.sync_copy(rows[0], acc.at[pl.ds(row0 + k * CH4, CH4)])
    plsc.subcore_barrier()

    coff = c * N

    for phase in range(NPHASE):
        crow = s * NCH4 + phase * HALF
        pltpu.sync_copy(src4_hbm.at[pl.ds(crow, HALF)], isrc)
        pltpu.sync_copy(dst4_hbm.at[pl.ds(crow, HALF)], idst)

        def offadd(r, carry):
            for j in range(CH4 // LANES):
                sl = pl.ds(j * LANES, LANES)
                isrc[r, sl] = isrc[r, sl] + coff
            return carry

        lax.fori_loop(0, HALF, offadd, 0)

        for b in range(NBUF):
            pltpu.async_copy(table_hbm.at[isrc.at[b]], rows[b], g[b])

        def body(m, carry):
            for b in range(NBUF):
                k = NBUF * m + b
                pltpu.make_async_copy(
                    table_hbm.at[isrc.at[k]], rows[b], g[b]).wait()
                pltpu.async_copy(rows[b], acc.at[idst.at[k]], s_sem[b],
                                 add=True)

                @pl.when(k + NBUF < HALF)
                def _():
                    pltpu.make_async_copy(
                        rows[b], acc.at[idst.at[k]], s_sem[b]).wait()
                    pltpu.async_copy(
                        table_hbm.at[isrc.at[k + NBUF]], rows[b], g[b])

            return carry

        lax.fori_loop(0, HALF // NBUF, body, 0)
        for b in range(NBUF):
            pltpu.make_async_copy(rows[b], acc.at[idst.at[0]], s_sem[b]).wait()
    plsc.subcore_barrier()

    for k in range(ROWS_PER_TILE // CHUNK):
        pltpu.sync_copy(acc.at[pl.ds(row0 + k * CHUNK, CHUNK)],
                        out_hbm.at[pl.ds(c * NR + row0 + k * CHUNK, CHUNK)])


def _dis_from_counts(counts_ref):
    cnt = counts_ref[0, :, 0:1] + counts_ref[1, :, 0:1]
    return lax.rsqrt(cnt + 1.0)


def _mm_scale_body(counts_ref, x_ref, w_ref, out_ref):
    dis = _dis_from_counts(counts_ref)
    p = jnp.dot(x_ref[...], w_ref[...], preferred_element_type=jnp.float32) * dis
    out_ref[0] = p[:, :T]
    out_ref[1] = p[:, T:]


def _finish_stats_body(counts_ref, s_ref, p_ref, b_ref, a_ref, stats_ref):
    dis = _dis_from_counts(counts_ref)
    a = jnp.concatenate([s_ref[0] + p_ref[0], s_ref[1] + p_ref[1]], axis=1)
    a = a * dis + b_ref[...]
    a_ref[...] = a

    @pl.when(pl.program_id(0) == 0)
    def _():
        stats_ref[...] = jnp.zeros_like(stats_ref)

    stats_ref[0:1, :] = stats_ref[0:1, :] + jnp.sum(a, axis=0, keepdims=True)
    stats_ref[1:2, :] = stats_ref[1:2, :] + jnp.sum(a * a, axis=0, keepdims=True)


def _bn_mm_body(counts_ref, a_ref, stats_ref, g_ref, be_ref, w_ref, out_ref):
    dis = _dis_from_counts(counts_ref)
    m = stats_ref[0:1, :] / N
    v = stats_ref[1:2, :] / N - m * m
    h = (a_ref[...] - m) * lax.rsqrt(v + 1e-5) * g_ref[...] + be_ref[...]
    h = jnp.maximum(h, 0.0)
    p = jnp.dot(h, w_ref[...], preferred_element_type=jnp.float32) * dis
    out_ref[0] = p[:, :T]
    out_ref[1] = p[:, T:]


def _pool_steps(counts_ref, s_ref, p_ref, b_ref, batch_ref, psum_ref, pcnt_ref):
    dis = _dis_from_counts(counts_ref)
    a = jnp.concatenate([s_ref[0] + p_ref[0], s_ref[1] + p_ref[1]], axis=1)
    h = jnp.maximum(a * dis + b_ref[...], 0.0)
    oh = (batch_ref[...] == lax.broadcasted_iota(jnp.int32, (1, G), 1))
    oh = oh.astype(jnp.float32)

    @pl.when(pl.program_id(0) == 0)
    def _():
        psum_ref[...] = jnp.zeros_like(psum_ref)
        pcnt_ref[...] = jnp.zeros_like(pcnt_ref)

    dn = (((0,), (0,)), ((), ()))
    psum_ref[...] = psum_ref[...] + lax.dot_general(
        oh, h, dn, preferred_element_type=jnp.float32)
    pcnt_ref[...] = pcnt_ref[...] + lax.dot_general(
        oh, jnp.ones((BR, 8), jnp.float32), dn, preferred_element_type=jnp.float32)


def _final_step(psum_ref, pcnt_ref, wl_ref, bl_ref, out_ref):
    pooled = psum_ref[...] / jnp.maximum(pcnt_ref[:, 0:1], 1.0)
    out_ref[...] = jnp.dot(pooled, wl_ref[...],
                           preferred_element_type=jnp.float32) + bl_ref[...]


NB = N // BR

_counts_spec = pl.BlockSpec((2, BR, LANES), lambda i: (0, i, 0))
_half_spec = pl.BlockSpec((2, BR, T), lambda i: (0, i, 0))

_finish_stats = pl.pallas_call(
    _finish_stats_body,
    grid=(NB,),
    in_specs=[_counts_spec, _half_spec, _half_spec,
              pl.BlockSpec((1, H), lambda i: (0, 0))],
    out_specs=[pl.BlockSpec((BR, H), lambda i: (i, 0)),
               pl.BlockSpec((2, H), lambda i: (0, 0))],
    out_shape=[jax.ShapeDtypeStruct((N, H), jnp.float32),
               jax.ShapeDtypeStruct((2, H), jnp.float32)],
)

_bn_mm = pl.pallas_call(
    _bn_mm_body,
    grid=(NB,),
    in_specs=[_counts_spec, pl.BlockSpec((BR, H), lambda i: (i, 0)),
              pl.BlockSpec((2, H), lambda i: (0, 0)),
              pl.BlockSpec((1, H), lambda i: (0, 0)),
              pl.BlockSpec((1, H), lambda i: (0, 0)),
              pl.BlockSpec((H, H), lambda i: (0, 0))],
    out_specs=_half_spec,
    out_shape=jax.ShapeDtypeStruct((2, N, T), jnp.float32),
)

_mm_scale = pl.pallas_call(
    _mm_scale_body,
    grid=(NB,),
    in_specs=[_counts_spec, pl.BlockSpec((BR, D), lambda i: (i, 0)),
              pl.BlockSpec((D, H), lambda i: (0, 0))],
    out_specs=_half_spec,
    out_shape=jax.ShapeDtypeStruct((2, N, T), jnp.float32),
)

_pool = pl.pallas_call(
    lambda counts_ref, s_ref, p_ref, b_ref, batch_ref, psum_ref, pcnt_ref:
        _pool_steps(counts_ref, s_ref, p_ref, b_ref, batch_ref, psum_ref, pcnt_ref),
    grid=(NB,),
    in_specs=[_counts_spec, _half_spec, _half_spec,
              pl.BlockSpec((1, H), lambda i: (0, 0)),
              pl.BlockSpec((BR, 1), lambda i: (i, 0))],
    out_specs=[pl.BlockSpec((G, H), lambda i: (0, 0)),
               pl.BlockSpec((G, 8), lambda i: (0, 0))],
    out_shape=[jax.ShapeDtypeStruct((G, H), jnp.float32),
               jax.ShapeDtypeStruct((G, 8), jnp.float32)],
)

_final = pl.pallas_call(
    lambda psum_ref, pcnt_ref, wl_ref, bl_ref, out_ref:
        _final_step(psum_ref, pcnt_ref, wl_ref, bl_ref, out_ref),
    grid=(1,),
    in_specs=[pl.BlockSpec((G, H), lambda i: (0, 0)),
              pl.BlockSpec((G, 8), lambda i: (0, 0)),
              pl.BlockSpec((H, T), lambda i: (0, 0)),
              pl.BlockSpec((1, T), lambda i: (0, 0))],
    out_specs=pl.BlockSpec((G, T), lambda i: (0, 0)),
    out_shape=jax.ShapeDtypeStruct((G, T), jnp.float32),
)

def kernel(x, edge_index, batch, W1, b1, W2, b2, W3, b3, g1, be1, g2, be2, Wl, bl):
    src = edge_index[0]
    dst = edge_index[1]
    pad = E_PAD - src.shape[0]
    src_p = jnp.concatenate([src, jnp.zeros((pad,), src.dtype)])
    dst_p = jnp.concatenate([dst, jnp.full((pad,), GARBAGE, dst.dtype)])
    dst2 = dst_p.reshape(E_PAD // CHUNK, CHUNK)
    src4 = src_p.reshape(E_PAD // CH4, CH4)
    dst4 = dst_p.reshape(E_PAD // CH4, CH4)

    counts = _count_kernel(dst2).reshape(NC, NR, LANES)

    def spmm(p):
        return _spmm_kernel(p.reshape(NC * N, T), src4, dst4).reshape(NC, NR, T)

    b1r, b2r, b3r = b1.reshape(1, H), b2.reshape(1, H), b3.reshape(1, H)

    p1 = _mm_scale(counts, x, W1)
    s1 = spmm(p1)
    a1, st1 = _finish_stats(counts, s1, p1, b1r)
    p2 = _bn_mm(counts, a1, st1, g1.reshape(1, H), be1.reshape(1, H), W2)
    s2 = spmm(p2)
    a2, st2 = _finish_stats(counts, s2, p2, b2r)
    p3 = _bn_mm(counts, a2, st2, g2.reshape(1, H), be2.reshape(1, H), W3)
    s3 = spmm(p3)
    psum, pcnt = _pool(counts, s3, p3, b3r, batch.reshape(N, 1))
    return _final(psum, pcnt, Wl, bl.reshape(1, T))


# speedup vs baseline: 1.1041x; 1.0945x over previous
import functools

import jax
import jax.numpy as jnp
from jax import lax
from jax.experimental import pallas as pl
from jax.experimental.pallas import tpu as pltpu
from jax.experimental.pallas import tpu_sc as plsc

N = 10000
D = 256
H = 256
T = 128
G = 64

NC = 2
NS = 16
LANES = 16

CHUNK = 128
E_PAD = NC * NS * CHUNK * 40
GARBAGE = N
NR = 10240
ROWS_PER_TILE = NR // NS

BR = 2000

_sc_mesh = plsc.VectorSubcoreMesh(core_axis_name="c", subcore_axis_name="s")


NCH_CNT = E_PAD // (NC * NS) // CHUNK
NBUF = 2
CH4 = 128
NCH4 = E_PAD // NS // CH4
NPHASE = 2


@functools.partial(
    pl.kernel,
    out_type=jax.ShapeDtypeStruct((NC * NR, LANES), jnp.float32),
    mesh=_sc_mesh,
    scratch_types=[
        pltpu.VMEM((NCH_CNT, CHUNK), jnp.int32),
        pltpu.VMEM((CHUNK, LANES), jnp.float32),
        pltpu.VMEM((CHUNK, LANES), jnp.float32),
        pltpu.VMEM_SHARED((NR, LANES), jnp.float32),
        pltpu.SemaphoreType.DMA,
    ],
)
def _count_kernel(dst2_hbm, out_hbm, idx_v, ones_v, zero_v, acc, sem):
    c = lax.axis_index("c")
    s = lax.axis_index("s")
    zvec = jnp.zeros((LANES,), jnp.float32)
    onevec = jnp.where(lax.iota(jnp.int32, LANES) == 0, 1.0, 0.0)

    def fill(r, carry):
        ones_v[r, :] = onevec
        zero_v[r, :] = zvec
        return carry

    lax.fori_loop(0, CHUNK, fill, 0)

    crow = (c * NS + s) * NCH_CNT
    pltpu.sync_copy(dst2_hbm.at[pl.ds(crow, NCH_CNT)], idx_v)

    row0 = s * ROWS_PER_TILE
    for k in range(ROWS_PER_TILE // CHUNK):
        pltpu.sync_copy(zero_v, acc.at[pl.ds(row0 + k * CHUNK, CHUNK)])
    plsc.subcore_barrier()

    def body(k, carry):
        pltpu.async_copy(ones_v, acc.at[idx_v.at[k]], sem, add=True)
        return carry

    lax.fori_loop(0, NCH_CNT, body, 0)

    def drain(k, carry):
        pltpu.make_async_copy(ones_v, acc.at[idx_v.at[0]], sem).wait()
        return carry

    lax.fori_loop(0, NCH_CNT, drain, 0)
    plsc.subcore_barrier()

    for k in range(ROWS_PER_TILE // CHUNK):
        pltpu.sync_copy(acc.at[pl.ds(row0 + k * CHUNK, CHUNK)],
                        out_hbm.at[pl.ds(c * NR + row0 + k * CHUNK, CHUNK)])


@functools.partial(
    pl.kernel,
    out_type=jax.ShapeDtypeStruct((NC * NR, T), jnp.float32),
    mesh=_sc_mesh,
    scratch_types=[
        pltpu.VMEM((NCH4 // NPHASE, CH4), jnp.int32),
        pltpu.VMEM((NCH4 // NPHASE, CH4), jnp.int32),
        [pltpu.VMEM((CH4, T), jnp.float32) for _ in range(NBUF)],
        pltpu.VMEM_SHARED((NR, T), jnp.float32),
        [pltpu.SemaphoreType.DMA for _ in range(NBUF)],
        [pltpu.SemaphoreType.DMA for _ in range(NBUF)],
    ],
)
def _spmm_kernel(table_hbm, src4_hbm, dst4_hbm, out_hbm,
                 isrc, idst, rows, acc, g, s_sem):
    c = lax.axis_index("c")
    s = lax.axis_index("s")
    zvec = jnp.zeros((LANES,), jnp.float32)
    HALF = NCH4 // NPHASE

    def zfill(r, carry):
        for j in range(T // LANES):
            rows[0][r, pl.ds(j * LANES, LANES)] = zvec
        return carry

    lax.fori_loop(0, CH4, zfill, 0)

    row0 = s * ROWS_PER_TILE
    for k in range(ROWS_PER_TILE // CH4):
        pltpu.sync_copy(rows[0], acc.at[pl.ds(row0 + k * CH4, CH4)])
    plsc.subcore_barrier()

    coff = c * N

    for phase in range(NPHASE):
        crow = s * NCH4 + phase * HALF
        pltpu.sync_copy(src4_hbm.at[pl.ds(crow, HALF)], isrc)
        pltpu.sync_copy(dst4_hbm.at[pl.ds(crow, HALF)], idst)

        def offadd(r, carry):
            for j in range(CH4 // LANES):
                sl = pl.ds(j * LANES, LANES)
                isrc[r, sl] = isrc[r, sl] + coff
            return carry

        lax.fori_loop(0, HALF, offadd, 0)

        for b in range(NBUF):
            pltpu.async_copy(table_hbm.at[isrc.at[b]], rows[b], g[b])

        def body(m, carry):
            for b in range(NBUF):
                k = NBUF * m + b
                pltpu.make_async_copy(
                    table_hbm.at[isrc.at[k]], rows[b], g[b]).wait()
                pltpu.async_copy(rows[b], acc.at[idst.at[k]], s_sem[b],
                                 add=True)

                @pl.when(k + NBUF < HALF)
                def _():
                    pltpu.make_async_copy(
                        rows[b], acc.at[idst.at[k]], s_sem[b]).wait()
                    pltpu.async_copy(
                        table_hbm.at[isrc.at[k + NBUF]], rows[b], g[b])

            return carry

        lax.fori_loop(0, HALF // NBUF, body, 0)
        for b in range(NBUF):
            pltpu.make_async_copy(rows[b], acc.at[idst.at[0]], s_sem[b]).wait()
    plsc.subcore_barrier()

    for k in range(ROWS_PER_TILE // CHUNK):
        pltpu.sync_copy(acc.at[pl.ds(row0 + k * CHUNK, CHUNK)],
                        out_hbm.at[pl.ds(c * NR + row0 + k * CHUNK, CHUNK)])


def _dis_from_counts(counts_ref):
    cnt = counts_ref[0, :, 0:1] + counts_ref[1, :, 0:1]
    return lax.rsqrt(cnt + 1.0)


def _mm_scale_body(counts_ref, x_ref, w_ref, out_ref):
    dis = _dis_from_counts(counts_ref)
    p = jnp.dot(x_ref[...], w_ref[...], preferred_element_type=jnp.float32) * dis
    out_ref[0] = p[:, :T]
    out_ref[1] = p[:, T:]


def _finish_stats_body(counts_ref, s_ref, p_ref, b_ref, a_ref, stats_ref):
    dis = _dis_from_counts(counts_ref)
    a = jnp.concatenate([s_ref[0] + p_ref[0], s_ref[1] + p_ref[1]], axis=1)
    a = a * dis + b_ref[...]
    a_ref[...] = a

    @pl.when(pl.program_id(0) == 0)
    def _():
        stats_ref[...] = jnp.zeros_like(stats_ref)

    stats_ref[0:1, :] = stats_ref[0:1, :] + jnp.sum(a, axis=0, keepdims=True)
    stats_ref[1:2, :] = stats_ref[1:2, :] + jnp.sum(a * a, axis=0, keepdims=True)


def _bn_mm_body(counts_ref, a_ref, stats_ref, g_ref, be_ref, w_ref, out_ref):
    dis = _dis_from_counts(counts_ref)
    m = stats_ref[0:1, :] / N
    v = stats_ref[1:2, :] / N - m * m
    h = (a_ref[...] - m) * lax.rsqrt(v + 1e-5) * g_ref[...] + be_ref[...]
    h = jnp.maximum(h, 0.0)
    p = jnp.dot(h, w_ref[...], preferred_element_type=jnp.float32) * dis
    out_ref[0] = p[:, :T]
    out_ref[1] = p[:, T:]


def _pool_steps(counts_ref, s_ref, p_ref, b_ref, batch_ref, wl_ref, bl_ref,
                out_ref, psum_buf, pcnt_buf):
    dis = _dis_from_counts(counts_ref)
    a = jnp.concatenate([s_ref[0] + p_ref[0], s_ref[1] + p_ref[1]], axis=1)
    h = jnp.maximum(a * dis + b_ref[...], 0.0)
    oh = (batch_ref[...] == lax.broadcasted_iota(jnp.int32, (1, G), 1))
    oh = oh.astype(jnp.float32)

    @pl.when(pl.program_id(0) == 0)
    def _():
        psum_buf[...] = jnp.zeros_like(psum_buf)
        pcnt_buf[...] = jnp.zeros_like(pcnt_buf)

    dn = (((0,), (0,)), ((), ()))
    psum_buf[...] = psum_buf[...] + lax.dot_general(
        oh, h, dn, preferred_element_type=jnp.float32)
    pcnt_buf[...] = pcnt_buf[...] + lax.dot_general(
        oh, jnp.ones((BR, 8), jnp.float32), dn, preferred_element_type=jnp.float32)

    pooled = psum_buf[...] / jnp.maximum(pcnt_buf[:, 0:1], 1.0)
    out_ref[...] = jnp.dot(pooled, wl_ref[...],
                           preferred_element_type=jnp.float32) + bl_ref[...]


NB = N // BR

_counts_spec = pl.BlockSpec((2, BR, LANES), lambda i: (0, i, 0))
_half_spec = pl.BlockSpec((2, BR, T), lambda i: (0, i, 0))

_finish_stats = pl.pallas_call(
    _finish_stats_body,
    grid=(NB,),
    in_specs=[_counts_spec, _half_spec, _half_spec,
              pl.BlockSpec((1, H), lambda i: (0, 0))],
    out_specs=[pl.BlockSpec((BR, H), lambda i: (i, 0)),
               pl.BlockSpec((2, H), lambda i: (0, 0))],
    out_shape=[jax.ShapeDtypeStruct((N, H), jnp.float32),
               jax.ShapeDtypeStruct((2, H), jnp.float32)],
)

_bn_mm = pl.pallas_call(
    _bn_mm_body,
    grid=(NB,),
    in_specs=[_counts_spec, pl.BlockSpec((BR, H), lambda i: (i, 0)),
              pl.BlockSpec((2, H), lambda i: (0, 0)),
              pl.BlockSpec((1, H), lambda i: (0, 0)),
              pl.BlockSpec((1, H), lambda i: (0, 0)),
              pl.BlockSpec((H, H), lambda i: (0, 0))],
    out_specs=_half_spec,
    out_shape=jax.ShapeDtypeStruct((2, N, T), jnp.float32),
)

_mm_scale = pl.pallas_call(
    _mm_scale_body,
    grid=(NB,),
    in_specs=[_counts_spec, pl.BlockSpec((BR, D), lambda i: (i, 0)),
              pl.BlockSpec((D, H), lambda i: (0, 0))],
    out_specs=_half_spec,
    out_shape=jax.ShapeDtypeStruct((2, N, T), jnp.float32),
)

_pool = pl.pallas_call(
    _pool_steps,
    grid=(NB,),
    in_specs=[_counts_spec, _half_spec, _half_spec,
              pl.BlockSpec((1, H), lambda i: (0, 0)),
              pl.BlockSpec((BR, 1), lambda i: (i, 0)),
              pl.BlockSpec((H, T), lambda i: (0, 0)),
              pl.BlockSpec((1, T), lambda i: (0, 0))],
    out_specs=pl.BlockSpec((G, T), lambda i: (0, 0)),
    out_shape=jax.ShapeDtypeStruct((G, T), jnp.float32),
    scratch_shapes=[pltpu.VMEM((G, H), jnp.float32),
                    pltpu.VMEM((G, 8), jnp.float32)],
)


def kernel(x, edge_index, batch, W1, b1, W2, b2, W3, b3, g1, be1, g2, be2, Wl, bl):
    src = edge_index[0]
    dst = edge_index[1]
    pad = E_PAD - src.shape[0]
    src_p = jnp.concatenate([src, jnp.zeros((pad,), src.dtype)])
    dst_p = jnp.concatenate([dst, jnp.full((pad,), GARBAGE, dst.dtype)])
    dst2 = dst_p.reshape(E_PAD // CHUNK, CHUNK)
    src4 = src_p.reshape(E_PAD // CH4, CH4)
    dst4 = dst_p.reshape(E_PAD // CH4, CH4)

    counts = _count_kernel(dst2).reshape(NC, NR, LANES)

    def spmm(p):
        return _spmm_kernel(p.reshape(NC * N, T), src4, dst4).reshape(NC, NR, T)

    b1r, b2r, b3r = b1.reshape(1, H), b2.reshape(1, H), b3.reshape(1, H)

    p1 = _mm_scale(counts, x, W1)
    s1 = spmm(p1)
    a1, st1 = _finish_stats(counts, s1, p1, b1r)
    p2 = _bn_mm(counts, a1, st1, g1.reshape(1, H), be1.reshape(1, H), W2)
    s2 = spmm(p2)
    a2, st2 = _finish_stats(counts, s2, p2, b2r)
    p3 = _bn_mm(counts, a2, st2, g2.reshape(1, H), be2.reshape(1, H), W3)
    s3 = spmm(p3)
    return _pool(counts, s3, p3, b3r, batch.reshape(N, 1), Wl,
                 bl.reshape(1, T))
